# Initial kernel scaffold; baseline (speedup 1.0000x reference)
import jax
import jax.numpy as jnp
from jax.experimental import pallas as pl


def kernel(user_ids, item_ids, edge_index, user_table, item_table, W1, b1, W2, b2):
    raise NotImplementedError("write your pallas kernel here")


# speedup vs baseline: 8.4745x; 8.4745x over previous
import functools

import jax
import jax.numpy as jnp
from jax import lax
from jax.experimental import pallas as pl
from jax.experimental.pallas import tpu as pltpu
from jax.experimental.pallas import tpu_sc as plsc

N_USERS = 2000
N_ITEMS = 8000
N = 10000
D = 256
DH = 128
NLAYER = 3
E = 160000
BATCH = 4096

NTILE = 16
EPT = E // NTILE
CHUNK = 80
NCHUNK = EPT // CHUNK
NPAD = 10240
RPT = NPAD // NTILE
PPT = BATCH // 32

_MESH = plsc.VectorSubcoreMesh(core_axis_name="c", subcore_axis_name="s",
                               num_cores=2, num_subcores=NTILE)


@functools.partial(
    pl.kernel,
    out_type=jax.ShapeDtypeStruct((NPAD, 16), jnp.float32),
    mesh=_MESH,
    compiler_params=pltpu.CompilerParams(use_tc_tiling_on_sc=False),
    scratch_types=[
        pltpu.VMEM((EPT,), jnp.int32),
        pltpu.VMEM((CHUNK,), jnp.int32),
        pltpu.VMEM((CHUNK, 16), jnp.float32),
        pltpu.VMEM_SHARED((NPAD, 16), jnp.float32),
    ],
)
def _deg_kernel(row1_hbm, ones_hbm, zeros8_hbm, deg8_hbm, row_v, rbuf, ones_v,
                acc):
    c = lax.axis_index("c")
    s = lax.axis_index("s")
    pltpu.sync_copy(zeros8_hbm, acc.at[pl.ds(s * RPT, RPT)])
    pltpu.sync_copy(row1_hbm.at[pl.ds(s * EPT, EPT)], row_v)
    pltpu.sync_copy(ones_hbm, ones_v)
    plsc.subcore_barrier()

    def body(g, carry):
        def cp(k, c2):
            rbuf[pl.ds(k * 16, 16)] = row_v[pl.ds(g * CHUNK + k * 16, 16)]
            return c2

        lax.fori_loop(0, CHUNK // 16, cp, 0)
        pltpu.sync_copy(ones_v, acc.at[rbuf], add=True)
        return carry

    lax.fori_loop(0, NCHUNK, body, 0)
    plsc.subcore_barrier()

    @pl.when(c == 0)
    def _():
        pltpu.sync_copy(acc.at[pl.ds(s * RPT, RPT)],
                        deg8_hbm.at[pl.ds(s * RPT, RPT)])


@functools.partial(
    pl.kernel,
    out_type=jax.ShapeDtypeStruct((2, NPAD, DH), jnp.float32),
    mesh=_MESH,
    compiler_params=pltpu.CompilerParams(use_tc_tiling_on_sc=False),
    scratch_types=[
        pltpu.VMEM((EPT,), jnp.int32),
        pltpu.VMEM((EPT,), jnp.int32),
        pltpu.VMEM((CHUNK,), jnp.int32),
        pltpu.VMEM((CHUNK,), jnp.int32),
        pltpu.VMEM((CHUNK, DH), jnp.float32),
        pltpu.VMEM((CHUNK, DH), jnp.float32),
        pltpu.SemaphoreType.DMA,
        pltpu.SemaphoreType.DMA,
        pltpu.VMEM_SHARED((NPAD, DH), jnp.float32),
    ],
)
def _spmm_kernel(tab_hbm, col1_hbm, row1_hbm, zdh_hbm, out_hbm,
                 col_v, row_v, rbuf0, rbuf1, gb0, gb1, sem0, sem1, acc):
    c = lax.axis_index("c")
    s = lax.axis_index("s")
    pltpu.sync_copy(zdh_hbm, acc.at[pl.ds(s * RPT, RPT)])
    pltpu.sync_copy(col1_hbm.at[pl.ds(s * EPT, EPT)], col_v)
    pltpu.sync_copy(row1_hbm.at[pl.ds(s * EPT, EPT)], row_v)

    def xform(i, carry):
        v = col_v[pl.ds(i * 16, 16)]
        col_v[pl.ds(i * 16, 16)] = v * 2 + c
        return carry

    lax.fori_loop(0, EPT // 16, xform, 0)
    plsc.subcore_barrier()

    def start(g, gb, sem):
        pltpu.async_copy(tab_hbm.at[col_v.at[pl.ds(g * CHUNK, CHUNK)]],
                         gb, sem)

    def finish(g, gb, rbuf, sem):
        pltpu.make_async_copy(tab_hbm.at[col_v.at[pl.ds(g * CHUNK, CHUNK)]],
                              gb, sem).wait()

        def cp(k, c2):
            rbuf[pl.ds(k * 16, 16)] = row_v[pl.ds(g * CHUNK + k * 16, 16)]
            return c2

        lax.fori_loop(0, CHUNK // 16, cp, 0)
        pltpu.sync_copy(gb, acc.at[rbuf], add=True)

    start(0, gb0, sem0)

    def body(m, carry):
        start(2 * m + 1, gb1, sem1)
        finish(2 * m, gb0, rbuf0, sem0)
        start(2 * m + 2, gb0, sem0)
        finish(2 * m + 1, gb1, rbuf1, sem1)
        return carry

    lax.fori_loop(0, (NCHUNK - 1) // 2, body, 0)
    finish(NCHUNK - 1, gb0, rbuf0, sem0)
    plsc.subcore_barrier()
    pltpu.sync_copy(acc.at[pl.ds(s * RPT, RPT)],
                    out_hbm.at[c, pl.ds(s * RPT, RPT)])


@functools.partial(
    pl.kernel,
    out_type=(jax.ShapeDtypeStruct((4, BATCH, D), jnp.float32),
              jax.ShapeDtypeStruct((4, BATCH, D), jnp.float32)),
    mesh=_MESH,
    compiler_params=pltpu.CompilerParams(use_tc_tiling_on_sc=False),
    scratch_types=[
        pltpu.VMEM((PPT,), jnp.int32),
        pltpu.VMEM((PPT,), jnp.int32),
        pltpu.VMEM((PPT, D), jnp.float32),
        pltpu.VMEM((PPT, D), jnp.float32),
        pltpu.SemaphoreType.DMA,
        pltpu.SemaphoreType.DMA,
    ],
)
def _pair_kernel(e0, e1, e2, e3, uid_hbm, vid_hbm, u_out, v_out,
                 uidx, vidx, gu, gv, semu, semv):
    c = lax.axis_index("c")
    s = lax.axis_index("s")
    w = s * 2 + c
    base = w * PPT
    pltpu.sync_copy(uid_hbm.at[pl.ds(base, PPT)], uidx)
    pltpu.sync_copy(vid_hbm.at[pl.ds(base, PPT)], vidx)

    def off(k, carry):
        vidx[pl.ds(k * 16, 16)] = vidx[pl.ds(k * 16, 16)] + N_USERS
        return carry

    lax.fori_loop(0, PPT // 16, off, 0)
    for tab_i, tab in enumerate((e0, e1, e2, e3)):
        cu = pltpu.async_copy(tab.at[uidx], gu, semu)
        cv = pltpu.async_copy(tab.at[vidx], gv, semv)
        cu.wait()
        pltpu.sync_copy(gu, u_out.at[tab_i, pl.ds(base, PPT)])
        cv.wait()
        pltpu.sync_copy(gv, v_out.at[tab_i, pl.ds(base, PPT)])


_BLK = 400


def _dis(deg_ref):
    deg = deg_ref[:, 0:1]
    return jnp.where(deg > 0.0, lax.rsqrt(deg), 0.0)


def _prep_body(deg_ref, emb_ref, es_ref):
    es_ref[...] = emb_ref[...] * _dis(deg_ref)


_prep_call = pl.pallas_call(
    _prep_body,
    grid=(N // _BLK,),
    in_specs=[pl.BlockSpec((_BLK, 16), lambda i: (i, 0)),
              pl.BlockSpec((_BLK, D), lambda i: (i, 0))],
    out_specs=pl.BlockSpec((_BLK, D), lambda i: (i, 0)),
    out_shape=jax.ShapeDtypeStruct((N, D), jnp.float32),
)


def _layer_body(deg_ref, emb_ref, a_ref, w1_ref, b1_ref,
                w2_ref, b2_ref, out_ref, es_ref):
    dis = _dis(deg_ref)
    emb = emb_ref[...]
    a = a_ref[...]
    neigh = jnp.concatenate([a[0], a[1]], axis=1) * dis
    ego = jnp.dot(emb, w1_ref[...],
                  preferred_element_type=jnp.float32) + b1_ref[...]
    side = jnp.dot(neigh * (1.0 + emb), w2_ref[...],
                   preferred_element_type=jnp.float32) + b2_ref[...]
    x = ego + side
    x = jnp.where(x >= 0.0, x, 0.2 * x)
    nrm = jnp.sqrt(jnp.sum(x * x, axis=1, keepdims=True))
    x = x / jnp.maximum(nrm, 1e-12)
    out_ref[...] = x
    es_ref[...] = x * dis


_layer_call = pl.pallas_call(
    _layer_body,
    grid=(N // _BLK,),
    in_specs=[pl.BlockSpec((_BLK, 16), lambda i: (i, 0)),
              pl.BlockSpec((_BLK, D), lambda i: (i, 0)),
              pl.BlockSpec((2, _BLK, DH), lambda i: (0, i, 0)),
              pl.BlockSpec((D, D), lambda i: (0, 0)),
              pl.BlockSpec((1, D), lambda i: (0, 0)),
              pl.BlockSpec((D, D), lambda i: (0, 0)),
              pl.BlockSpec((1, D), lambda i: (0, 0))],
    out_specs=[pl.BlockSpec((_BLK, D), lambda i: (i, 0)),
               pl.BlockSpec((_BLK, D), lambda i: (i, 0))],
    out_shape=(jax.ShapeDtypeStruct((N, D), jnp.float32),
               jax.ShapeDtypeStruct((N, D), jnp.float32)),
)


_DBLK = 512


def _dot_body(u_ref, v_ref, o_ref):
    x = u_ref[...] * v_ref[...]
    r = jnp.sum(jnp.sum(x, axis=2), axis=0)
    o_ref[...] = r[:, None]


_dot_call = pl.pallas_call(
    _dot_body,
    grid=(BATCH // _DBLK,),
    in_specs=[pl.BlockSpec((4, _DBLK, D), lambda i: (0, i, 0)),
              pl.BlockSpec((4, _DBLK, D), lambda i: (0, i, 0))],
    out_specs=pl.BlockSpec((_DBLK, 1), lambda i: (i, 0)),
    out_shape=jax.ShapeDtypeStruct((BATCH, 1), jnp.float32),
)


def kernel(user_ids, item_ids, edge_index, user_table, item_table,
           W1, b1, W2, b2):
    row = edge_index[0].astype(jnp.int32)
    col = edge_index[1].astype(jnp.int32)
    emb0 = jnp.concatenate([user_table, item_table], axis=0)
    zeros8 = jnp.zeros((RPT, 16), jnp.float32)
    ones8 = jnp.ones((CHUNK, 16), jnp.float32)
    zdh = jnp.zeros((RPT, DH), jnp.float32)

    deg8 = _deg_kernel(row, ones8, zeros8)
    es = _prep_call(deg8, emb0)

    outs = [emb0]
    emb = emb0
    for i in range(NLAYER):
        acc2 = _spmm_kernel(es.reshape(2 * N, DH), col, row, zdh)
        out_i, es = _layer_call(deg8, emb, acc2,
                                W1[i], b1[i].reshape(1, D),
                                W2[i], b2[i].reshape(1, D))
        outs.append(out_i)
        emb = out_i

    uid = user_ids.astype(jnp.int32)
    vid = item_ids.astype(jnp.int32)
    u3, v3 = _pair_kernel(outs[0], outs[1], outs[2], outs[3], uid, vid)
    score = _dot_call(u3, v3)
    return score.reshape(BATCH)
